# baseline (speedup 1.0000x reference)
import functools

import jax
import jax.numpy as jnp
from jax import lax
from jax.experimental import pallas as pl
from jax.experimental.pallas import tpu as pltpu
from jax.experimental.pallas import tpu_sc as plsc

_NC = 2
_NS = 16
_NW = _NC * _NS
_C = 8
_RING = 3


def _sc_body(x_hbm, pe_hbm, out_hbm, pe_v, x_v, sem_pe, sem_ld, sem_st,
             *, B, S, D):
    wid = lax.axis_index("s") * _NC + lax.axis_index("c")
    ppw = S // _NW
    nch = ppw // _C
    base = wid * ppw

    def start_pe(g, sl):
        pltpu.async_copy(
            pe_hbm.at[pl.ds(base + g * _C, _C)], pe_v.at[sl], sem_pe.at[sl])

    def wait_pe(sl):
        pltpu.make_async_copy(
            pe_hbm.at[pl.ds(base, _C)], pe_v.at[sl], sem_pe.at[sl]).wait()

    def start_ld(g, sl):
        pltpu.async_copy(
            x_hbm.at[:, pl.ds(base + g * _C, _C)], x_v.at[sl], sem_ld.at[sl])

    def wait_ld(sl):
        pltpu.make_async_copy(
            x_hbm.at[:, pl.ds(base, _C)], x_v.at[sl], sem_ld.at[sl]).wait()

    def start_st(g, sl, b):
        pltpu.async_copy(
            x_v.at[sl, b], out_hbm.at[b, pl.ds(base + g * _C, _C)],
            sem_st.at[sl])

    def wait_st(sl, b):
        pltpu.make_async_copy(
            x_v.at[sl, b], out_hbm.at[b, pl.ds(base, _C)],
            sem_st.at[sl]).wait()

    def run_chunk(g, sl, prefetch_g, drain=True):
        wait_pe(sl)
        wait_ld(sl)
        for b in range(B):
            @plsc.parallel_loop(0, D, step=16, unroll=1)
            def _(i):
                for rw in range(_C):
                    plsc.addupdate(x_v.at[sl, b, rw, pl.ds(i, 16)],
                                   pe_v.at[sl][rw, pl.ds(i, 16)])

            start_st(g, sl, b)
        if prefetch_g is not None:
            sl2 = (sl + 2) % _RING
            if drain:
                for b in range(B):
                    wait_st(sl2, b)
            start_ld(prefetch_g, sl2)
            start_pe(prefetch_g, sl2)

    start_pe(0, 0)
    start_ld(0, 0)
    start_pe(1, 1)
    start_ld(1, 1)

    run_chunk(0, 0, 2, drain=False)
    run_chunk(1, 1, 3)
    run_chunk(2, 2, 4)

    n_groups = (nch - 5) // _RING

    def group_body(gi, _):
        g0 = 3 + gi * _RING
        for j in range(_RING):
            run_chunk(g0 + j, j, g0 + j + 2)
        return ()

    lax.fori_loop(0, n_groups, group_body, ())

    run_chunk(nch - 2, (nch - 2) % _RING, None)
    run_chunk(nch - 1, (nch - 1) % _RING, None)

    for g in range(nch - 3, nch):
        for b in range(B):
            wait_st(g % _RING, b)


def kernel(x, pe):
    B, S, D = x.shape

    mesh = plsc.VectorSubcoreMesh(core_axis_name="c", subcore_axis_name="s")
    k = pl.kernel(
        functools.partial(_sc_body, B=B, S=S, D=D),
        out_type=jax.ShapeDtypeStruct((B, S, D), jnp.float32),
        mesh=mesh,
        scratch_types=[
            pltpu.VMEM((_RING, _C, D), jnp.float32),
            pltpu.VMEM((_RING, B, _C, D), jnp.float32),
            pltpu.SemaphoreType.DMA((_RING,)),
            pltpu.SemaphoreType.DMA((_RING,)),
            pltpu.SemaphoreType.DMA((_RING,)),
        ],
    )
    return k(x, pe[:S])
